# baseline (speedup 1.0000x reference)
import functools

import jax
import jax.numpy as jnp
from jax import lax
from jax.experimental import pallas as pl
from jax.experimental.pallas import tpu as pltpu
from jax.experimental.pallas import tpu_sc as plsc

N_NODES = 10001
E = 160000
HID = 128
CUTOFF_UPPER = 4.5

N_PAD = 10240
NB = 256
EB = 1600
W = 40
PER_TILE = E // 32
N_WIN = PER_TILE // W
GW = 5
N_GROUP = N_WIN // GW
RING = 3
N_STRIPE = N_PAD // 16


def _silu(x):
    return x * (1.0 / (1.0 + jnp.exp(-x)))


def _bdot(a, b):
    return lax.dot_general(a.astype(jnp.bfloat16), b.astype(jnp.bfloat16),
                           (((1,), (1,)), ((), ())),
                           preferred_element_type=jnp.float32)


def _expand9(c, a, s, extra):
    return [
        c + s[0] + extra,
        a[0] + s[1] + extra,
        a[1] + s[2] + extra,
        -a[0] + s[1] + extra,
        c + s[3] + extra,
        a[2] + s[4] + extra,
        -a[1] + s[2] + extra,
        -a[2] + s[4] + extra,
        c - s[0] - s[3] + extra,
    ]


def _mat3mul(A, B):
    return [
        sum(A[3 * i + r] * B[3 * r + j] for r in range(3))
        for i in range(3) for j in range(3)
    ]


def _table_body(xt_ref, w0_ref, w1_ref, w2_ref, t_ref):
    x = [xt_ref[p] for p in range(9)]
    norm2 = sum(v * v for v in x) + 1.0
    inv = 1.0 / norm2
    x = [v * inv for v in x]
    ci = (x[0] + x[4] + x[8]) * (1.0 / 3.0)
    comps = [
        ci,
        0.5 * (x[1] - x[3]), 0.5 * (x[2] - x[6]), 0.5 * (x[5] - x[7]),
        x[0] - ci, 0.5 * (x[1] + x[3]), 0.5 * (x[2] + x[6]),
        x[4] - ci, 0.5 * (x[5] + x[7]),
    ]
    ws = [w0_ref[...], w1_ref[...], w2_ref[...]]
    wsel = [0, 1, 1, 1, 2, 2, 2, 2, 2]
    for p in range(9):
        t_ref[p] = _bdot(comps[p], ws[wsel[p]])


def _node_table(xt, w1_0, w1_1, w1_2):
    grid = N_PAD // NB
    return pl.pallas_call(
        _table_body,
        grid=(grid,),
        in_specs=[
            pl.BlockSpec((9, NB, HID), lambda i: (0, i, 0)),
            pl.BlockSpec((HID, HID), lambda i: (0, 0)),
            pl.BlockSpec((HID, HID), lambda i: (0, 0)),
            pl.BlockSpec((HID, HID), lambda i: (0, 0)),
        ],
        out_specs=pl.BlockSpec((9, NB, HID), lambda i: (0, i, 0)),
        out_shape=jax.ShapeDtypeStruct((9, N_PAD, HID), jnp.float32),
    )(xt, w1_0, w1_1, w1_2)


def _mlp_body(ea_ref, ew_ref, w0_ref, b0_ref, w1_ref, b1_ref,
              wk_ref, bk_ref, b1k_ref, f0_ref, f1_ref, f2_ref, g_ref):
    h = _silu(_bdot(ea_ref[...], w0_ref[...]) + b0_ref[...])
    h = _silu(_bdot(h, w1_ref[...]) + b1_ref[...])
    w = ew_ref[...]
    c = 0.5 * (jnp.cos(w * (jnp.pi / CUTOFF_UPPER)) + 1.0)
    c = jnp.where(w < CUTOFF_UPPER, c, 0.0)
    outs = [f0_ref, f1_ref, f2_ref]
    fks = []
    for k in range(3):
        fk = _silu(_bdot(h, wk_ref[k]) + bk_ref[k]) * c
        outs[k][...] = fk
        fks.append(fk)
    g_ref[...] = (fks[0] * b1k_ref[0] + fks[1] * b1k_ref[1]
                  + fks[2] * b1k_ref[2])


def _edge_mlp(edge_attr, edge_weight, w2_0, b2_0, w2_1, b2_1, wk, bk, b1k):
    grid = E // EB
    out = pl.pallas_call(
        _mlp_body,
        grid=(grid,),
        in_specs=[
            pl.BlockSpec((EB, 32), lambda i: (i, 0)),
            pl.BlockSpec((EB, 1), lambda i: (i, 0)),
            pl.BlockSpec((HID, 32), lambda i: (0, 0)),
            pl.BlockSpec((1, HID), lambda i: (0, 0)),
            pl.BlockSpec((2 * HID, HID), lambda i: (0, 0)),
            pl.BlockSpec((1, 2 * HID), lambda i: (0, 0)),
            pl.BlockSpec((3, HID, 2 * HID), lambda i: (0, 0, 0)),
            pl.BlockSpec((3, 1, HID), lambda i: (0, 0, 0)),
            pl.BlockSpec((3, 1, HID), lambda i: (0, 0, 0)),
        ],
        out_specs=[pl.BlockSpec((EB, HID), lambda i: (i, 0))] * 4,
        out_shape=[jax.ShapeDtypeStruct((E, HID), jnp.float32)] * 4,
    )(edge_attr, edge_weight, w2_0, b2_0, w2_1, b2_1, wk, bk, b1k)
    return out


def _sc_body(t, src_r, dst_r, f0, f1, f2, fg, zeros, out,
             srcall, dstall, rows, fac, acc,
             sem_ix, sem_g, sem_f, sem_s):
    c = lax.axis_index("c")
    s = lax.axis_index("s")
    e_base = c * (E // 2) + s * PER_TILE
    w_base = c * (E // 2 // W) + s * N_WIN

    def drain_one(slot):
        pltpu.make_async_copy(zeros.at[pl.ds(0, W)], rows.at[slot],
                              sem_s).wait()

    fsel = [f0, f1, f1, f1, f2, f2, f2, f2, f2, fg]

    for p in range(10):
        d1 = pltpu.async_copy(src_r.at[pl.ds(w_base, N_WIN)], srcall, sem_ix)
        d2 = pltpu.async_copy(dst_r.at[pl.ds(w_base, N_WIN)], dstall, sem_ix)

        @pl.when(s == 0)
        def _():
            pltpu.sync_copy(zeros, acc)

        d1.wait()
        d2.wait()
        plsc.subcore_barrier()

        def group(gi, carry):
            w0 = gi * GW

            @pl.when(gi > 0)
            def _():
                for r in range(RING):
                    drain_one(r)

            gds = []
            fds = []

            def fire(j):
                slot = j % RING
                e0 = e_base + (w0 + j) * W
                if p < 9:
                    gds.append(pltpu.async_copy(
                        t.at[p].at[srcall.at[w0 + j]], rows.at[slot], sem_g))
                    fds.append(pltpu.async_copy(
                        fsel[p].at[pl.ds(e0, W)], fac.at[slot], sem_f))
                else:
                    gds.append(pltpu.async_copy(
                        fsel[p].at[pl.ds(e0, W)], rows.at[slot], sem_g))

            for j in range(RING):
                fire(j)

            for j in range(GW):
                slot = j % RING
                gds[j].wait()
                if p < 9:
                    fds[j].wait()

                    @plsc.parallel_loop(0, W, unroll=2)
                    def _(e):
                        for q in range(8):
                            sl = pl.ds(16 * q, 16)
                            rows[slot, e, sl] = rows[slot, e, sl] * fac[slot, e, sl]
                pltpu.async_copy(rows.at[slot], acc.at[dstall.at[w0 + j]],
                                 sem_s, add=True)
                if j + RING < GW:
                    drain_one(slot)
                    fire(j + RING)
            return 0

        lax.fori_loop(0, N_GROUP, group, 0)
        for r in range(RING):
            drain_one(r)
        plsc.subcore_barrier()
        r0 = s * N_STRIPE
        pltpu.sync_copy(acc.at[pl.ds(r0, N_STRIPE)],
                        out.at[c].at[p].at[pl.ds(r0, N_STRIPE)])
        plsc.subcore_barrier()


def _sc_messages(t, src_r, dst_r, f0, f1, f2, fg, zeros):
    mesh = plsc.VectorSubcoreMesh(core_axis_name="c", subcore_axis_name="s")
    fn = pl.kernel(
        _sc_body,
        out_type=jax.ShapeDtypeStruct((2, 10, N_PAD, HID), jnp.float32),
        mesh=mesh,
        compiler_params=pltpu.CompilerParams(use_tc_tiling_on_sc=False),
        scratch_types=[
            pltpu.VMEM((N_WIN, W), jnp.int32),
            pltpu.VMEM((N_WIN, W), jnp.int32),
            pltpu.VMEM((RING, W, HID), jnp.float32),
            pltpu.VMEM((RING, W, HID), jnp.float32),
            pltpu.VMEM_SHARED((N_PAD, HID), jnp.float32),
            pltpu.SemaphoreType.DMA,
            pltpu.SemaphoreType.DMA,
            pltpu.SemaphoreType.DMA,
            pltpu.SemaphoreType.DMA,
        ],
    )
    return fn(t, src_r, dst_r, f0, f1, f2, fg, zeros)


def _final_body(xt_ref, t_ref, m_ref, w0_ref, w1_ref, w2_ref,
                bt1_ref, bt3_ref, out_ref):
    x = [xt_ref[p] for p in range(9)]
    norm2 = sum(v * v for v in x) + 1.0
    inv = 1.0 / norm2
    xn = [v * inv for v in x]

    bt1 = bt1_ref[...]
    t = [t_ref[p] for p in range(9)]
    Y = _expand9(t[0], t[1:4], t[4:9], bt1)

    m = [m_ref[0, p] + m_ref[1, p] for p in range(10)]
    M = _expand9(m[0], m[1:4], m[4:9], m[9])

    nf = [p + q for p, q in zip(_mat3mul(M, Y), _mat3mul(Y, M))]

    ci = (nf[0] + nf[4] + nf[8]) * (1.0 / 3.0)
    comps = [
        ci,
        0.5 * (nf[1] - nf[3]), 0.5 * (nf[2] - nf[6]), 0.5 * (nf[5] - nf[7]),
        nf[0] - ci, 0.5 * (nf[1] + nf[3]), 0.5 * (nf[2] + nf[6]),
        nf[4] - ci, 0.5 * (nf[5] + nf[7]),
    ]
    nrm = 1.0 / (sum(v * v for v in nf) + 1.0)
    comps = [v * nrm for v in comps]

    ws = [w0_ref[...], w1_ref[...], w2_ref[...]]
    wsel = [0, 1, 1, 1, 2, 2, 2, 2, 2]
    d = [_bdot(comps[p], ws[wsel[p]]) for p in range(9)]
    bt3 = bt3_ref[...]
    dX = _expand9(d[0], d[1:4], d[4:9], bt3)
    dX2 = _mat3mul(dX, dX)
    for p in range(9):
        out_ref[p] = xn[p] + dX[p] + dX2[p]


def _finalize(xt, t, m, w3_0, w3_1, w3_2, bt1, bt3):
    grid = N_PAD // NB
    return pl.pallas_call(
        _final_body,
        grid=(grid,),
        in_specs=[
            pl.BlockSpec((9, NB, HID), lambda i: (0, i, 0)),
            pl.BlockSpec((9, NB, HID), lambda i: (0, i, 0)),
            pl.BlockSpec((2, 10, NB, HID), lambda i: (0, 0, i, 0)),
            pl.BlockSpec((HID, HID), lambda i: (0, 0)),
            pl.BlockSpec((HID, HID), lambda i: (0, 0)),
            pl.BlockSpec((HID, HID), lambda i: (0, 0)),
            pl.BlockSpec((1, HID), lambda i: (0, 0)),
            pl.BlockSpec((1, HID), lambda i: (0, 0)),
        ],
        out_specs=pl.BlockSpec((9, NB, HID), lambda i: (0, i, 0)),
        out_shape=jax.ShapeDtypeStruct((9, N_PAD, HID), jnp.float32),
    )(xt, t, m, w3_0, w3_1, w3_2, bt1, bt3)


def kernel(X, edge_index, edge_weight, edge_attr,
           W1_0, b1_0, W1_1, b1_1, W1_2, b1_2,
           W3_0, b3_0, W3_1, b3_1, W3_2, b3_2,
           W2_0, b2_0, W2_1, b2_1, W2_2, b2_2):
    xt = jnp.transpose(X.reshape(N_NODES, HID, 9), (2, 0, 1))
    xt = jnp.pad(xt, ((0, 0), (0, N_PAD - N_NODES), (0, 0)))

    t = _node_table(xt, W1_0, W1_1, W1_2)

    wk = jnp.stack([W2_2[k::3, :] for k in range(3)])
    bk = jnp.stack([b2_2[k::3][None, :] for k in range(3)])
    b1k = jnp.stack([b1_0[None, :], b1_1[None, :], b1_2[None, :]])
    f0, f1, f2, fg = _edge_mlp(edge_attr, edge_weight[:, None],
                               W2_0, b2_0[None, :], W2_1, b2_1[None, :],
                               wk, bk, b1k)

    zeros = jnp.zeros((N_PAD, HID), jnp.float32)
    src_r = edge_index[1].reshape(E // W, W)
    dst_r = edge_index[0].reshape(E // W, W)
    m = _sc_messages(t, src_r, dst_r, f0, f1, f2, fg, zeros)

    bt1 = (b1_0 + b1_1 + b1_2)[None, :]
    bt3 = (b3_0 + b3_1 + b3_2)[None, :]
    xo = _finalize(xt, t, m, W3_0, W3_1, W3_2, bt1, bt3)

    xo = xo[:, :N_NODES, :]
    return jnp.transpose(xo, (1, 2, 0)).reshape(N_NODES, HID, 3, 3)


# speedup vs baseline: 1.0011x; 1.0011x over previous
import functools

import jax
import jax.numpy as jnp
from jax import lax
from jax.experimental import pallas as pl
from jax.experimental.pallas import tpu as pltpu
from jax.experimental.pallas import tpu_sc as plsc

N_NODES = 10001
E = 160000
HID = 128
CUTOFF_UPPER = 4.5

N_PAD = 10240
NB = 256
EB = 1600
W = 40
PER_TILE = E // 32
N_WIN = PER_TILE // W
GW = 5
N_GROUP = N_WIN // GW
RING = 3
N_STRIPE = N_PAD // 16


def _silu(x):
    return x * (1.0 / (1.0 + jnp.exp(-x)))


def _bdot(a, b):
    return lax.dot_general(a, b, (((1,), (1,)), ((), ())),
                           preferred_element_type=jnp.float32)


def _expand9(c, a, s, extra):
    return [
        c + s[0] + extra,
        a[0] + s[1] + extra,
        a[1] + s[2] + extra,
        -a[0] + s[1] + extra,
        c + s[3] + extra,
        a[2] + s[4] + extra,
        -a[1] + s[2] + extra,
        -a[2] + s[4] + extra,
        c - s[0] - s[3] + extra,
    ]


def _mat3mul(A, B):
    return [
        sum(A[3 * i + r] * B[3 * r + j] for r in range(3))
        for i in range(3) for j in range(3)
    ]


def _table_body(xt_ref, w0_ref, w1_ref, w2_ref, t_ref):
    x = [xt_ref[p] for p in range(9)]
    norm2 = sum(v * v for v in x) + 1.0
    inv = 1.0 / norm2
    x = [v * inv for v in x]
    ci = (x[0] + x[4] + x[8]) * (1.0 / 3.0)
    comps = [
        ci,
        0.5 * (x[1] - x[3]), 0.5 * (x[2] - x[6]), 0.5 * (x[5] - x[7]),
        x[0] - ci, 0.5 * (x[1] + x[3]), 0.5 * (x[2] + x[6]),
        x[4] - ci, 0.5 * (x[5] + x[7]),
    ]
    ws = [w0_ref[...], w1_ref[...], w2_ref[...]]
    wsel = [0, 1, 1, 1, 2, 2, 2, 2, 2]
    for p in range(9):
        t_ref[p] = _bdot(comps[p], ws[wsel[p]])


def _node_table(xt, w1_0, w1_1, w1_2):
    grid = N_PAD // NB
    return pl.pallas_call(
        _table_body,
        grid=(grid,),
        in_specs=[
            pl.BlockSpec((9, NB, HID), lambda i: (0, i, 0)),
            pl.BlockSpec((HID, HID), lambda i: (0, 0)),
            pl.BlockSpec((HID, HID), lambda i: (0, 0)),
            pl.BlockSpec((HID, HID), lambda i: (0, 0)),
        ],
        out_specs=pl.BlockSpec((9, NB, HID), lambda i: (0, i, 0)),
        out_shape=jax.ShapeDtypeStruct((9, N_PAD, HID), jnp.float32),
    )(xt, w1_0, w1_1, w1_2)


def _mlp_body(ea_ref, ew_ref, w0_ref, b0_ref, w1_ref, b1_ref,
              wk_ref, bk_ref, b1k_ref, f0_ref, f1_ref, f2_ref, g_ref):
    h = _silu(_bdot(ea_ref[...], w0_ref[...]) + b0_ref[...])
    h = _silu(_bdot(h, w1_ref[...]) + b1_ref[...])
    w = ew_ref[...]
    c = 0.5 * (jnp.cos(w * (jnp.pi / CUTOFF_UPPER)) + 1.0)
    c = jnp.where(w < CUTOFF_UPPER, c, 0.0)
    outs = [f0_ref, f1_ref, f2_ref]
    fks = []
    for k in range(3):
        fk = _silu(_bdot(h, wk_ref[k]) + bk_ref[k]) * c
        outs[k][...] = fk
        fks.append(fk)
    g_ref[...] = (fks[0] * b1k_ref[0] + fks[1] * b1k_ref[1]
                  + fks[2] * b1k_ref[2])


def _edge_mlp(edge_attr, edge_weight, w2_0, b2_0, w2_1, b2_1, wk, bk, b1k):
    grid = E // EB
    out = pl.pallas_call(
        _mlp_body,
        grid=(grid,),
        in_specs=[
            pl.BlockSpec((EB, 32), lambda i: (i, 0)),
            pl.BlockSpec((EB, 1), lambda i: (i, 0)),
            pl.BlockSpec((HID, 32), lambda i: (0, 0)),
            pl.BlockSpec((1, HID), lambda i: (0, 0)),
            pl.BlockSpec((2 * HID, HID), lambda i: (0, 0)),
            pl.BlockSpec((1, 2 * HID), lambda i: (0, 0)),
            pl.BlockSpec((3, HID, 2 * HID), lambda i: (0, 0, 0)),
            pl.BlockSpec((3, 1, HID), lambda i: (0, 0, 0)),
            pl.BlockSpec((3, 1, HID), lambda i: (0, 0, 0)),
        ],
        out_specs=[pl.BlockSpec((EB, HID), lambda i: (i, 0))] * 4,
        out_shape=[jax.ShapeDtypeStruct((E, HID), jnp.float32)] * 4,
    )(edge_attr, edge_weight, w2_0, b2_0, w2_1, b2_1, wk, bk, b1k)
    return out


def _sc_body(t, src_r, dst_r, f0, f1, f2, fg, zeros, out,
             srcall, dstall, rows, fac, acc,
             sem_ix, sem_g, sem_f, sem_s):
    c = lax.axis_index("c")
    s = lax.axis_index("s")
    e_base = c * (E // 2) + s * PER_TILE
    w_base = c * (E // 2 // W) + s * N_WIN

    def drain_one(slot):
        pltpu.make_async_copy(zeros.at[pl.ds(0, W)], rows.at[slot],
                              sem_s).wait()

    fsel = [f0, f1, f1, f1, f2, f2, f2, f2, f2, fg]

    for p in range(10):
        d1 = pltpu.async_copy(src_r.at[pl.ds(w_base, N_WIN)], srcall, sem_ix)
        d2 = pltpu.async_copy(dst_r.at[pl.ds(w_base, N_WIN)], dstall, sem_ix)

        @pl.when(s == 0)
        def _():
            pltpu.sync_copy(zeros, acc)

        d1.wait()
        d2.wait()
        plsc.subcore_barrier()

        def group(gi, carry):
            w0 = gi * GW

            @pl.when(gi > 0)
            def _():
                for r in range(RING):
                    drain_one(r)

            gds = []
            fds = []

            def fire(j):
                slot = j % RING
                e0 = e_base + (w0 + j) * W
                if p < 9:
                    gds.append(pltpu.async_copy(
                        t.at[p].at[srcall.at[w0 + j]], rows.at[slot], sem_g))
                    fds.append(pltpu.async_copy(
                        fsel[p].at[pl.ds(e0, W)], fac.at[slot], sem_f))
                else:
                    gds.append(pltpu.async_copy(
                        fsel[p].at[pl.ds(e0, W)], rows.at[slot], sem_g))

            for j in range(RING):
                fire(j)

            for j in range(GW):
                slot = j % RING
                gds[j].wait()
                if p < 9:
                    fds[j].wait()

                    @plsc.parallel_loop(0, W, unroll=2)
                    def _(e):
                        for q in range(8):
                            sl = pl.ds(16 * q, 16)
                            rows[slot, e, sl] = rows[slot, e, sl] * fac[slot, e, sl]
                pltpu.async_copy(rows.at[slot], acc.at[dstall.at[w0 + j]],
                                 sem_s, add=True)
                if j + RING < GW:
                    drain_one(slot)
                    fire(j + RING)
            return 0

        lax.fori_loop(0, N_GROUP, group, 0)
        for r in range(RING):
            drain_one(r)
        plsc.subcore_barrier()
        r0 = s * N_STRIPE
        pltpu.sync_copy(acc.at[pl.ds(r0, N_STRIPE)],
                        out.at[c].at[p].at[pl.ds(r0, N_STRIPE)])
        plsc.subcore_barrier()


def _sc_messages(t, src_r, dst_r, f0, f1, f2, fg, zeros):
    mesh = plsc.VectorSubcoreMesh(core_axis_name="c", subcore_axis_name="s")
    fn = pl.kernel(
        _sc_body,
        out_type=jax.ShapeDtypeStruct((2, 10, N_PAD, HID), jnp.float32),
        mesh=mesh,
        compiler_params=pltpu.CompilerParams(use_tc_tiling_on_sc=False),
        scratch_types=[
            pltpu.VMEM((N_WIN, W), jnp.int32),
            pltpu.VMEM((N_WIN, W), jnp.int32),
            pltpu.VMEM((RING, W, HID), jnp.float32),
            pltpu.VMEM((RING, W, HID), jnp.float32),
            pltpu.VMEM_SHARED((N_PAD, HID), jnp.float32),
            pltpu.SemaphoreType.DMA,
            pltpu.SemaphoreType.DMA,
            pltpu.SemaphoreType.DMA,
            pltpu.SemaphoreType.DMA,
        ],
    )
    return fn(t, src_r, dst_r, f0, f1, f2, fg, zeros)


def _final_body(xt_ref, t_ref, m_ref, w0_ref, w1_ref, w2_ref,
                bt1_ref, bt3_ref, out_ref):
    x = [xt_ref[p] for p in range(9)]
    norm2 = sum(v * v for v in x) + 1.0
    inv = 1.0 / norm2
    xn = [v * inv for v in x]

    bt1 = bt1_ref[...]
    t = [t_ref[p] for p in range(9)]
    Y = _expand9(t[0], t[1:4], t[4:9], bt1)

    m = [m_ref[0, p] + m_ref[1, p] for p in range(10)]
    M = _expand9(m[0], m[1:4], m[4:9], m[9])

    nf = [p + q for p, q in zip(_mat3mul(M, Y), _mat3mul(Y, M))]

    ci = (nf[0] + nf[4] + nf[8]) * (1.0 / 3.0)
    comps = [
        ci,
        0.5 * (nf[1] - nf[3]), 0.5 * (nf[2] - nf[6]), 0.5 * (nf[5] - nf[7]),
        nf[0] - ci, 0.5 * (nf[1] + nf[3]), 0.5 * (nf[2] + nf[6]),
        nf[4] - ci, 0.5 * (nf[5] + nf[7]),
    ]
    nrm = 1.0 / (sum(v * v for v in nf) + 1.0)
    comps = [v * nrm for v in comps]

    ws = [w0_ref[...], w1_ref[...], w2_ref[...]]
    wsel = [0, 1, 1, 1, 2, 2, 2, 2, 2]
    d = [_bdot(comps[p], ws[wsel[p]]) for p in range(9)]
    bt3 = bt3_ref[...]
    dX = _expand9(d[0], d[1:4], d[4:9], bt3)
    dX2 = _mat3mul(dX, dX)
    for p in range(9):
        out_ref[p] = xn[p] + dX[p] + dX2[p]


def _finalize(xt, t, m, w3_0, w3_1, w3_2, bt1, bt3):
    grid = N_PAD // NB
    return pl.pallas_call(
        _final_body,
        grid=(grid,),
        in_specs=[
            pl.BlockSpec((9, NB, HID), lambda i: (0, i, 0)),
            pl.BlockSpec((9, NB, HID), lambda i: (0, i, 0)),
            pl.BlockSpec((2, 10, NB, HID), lambda i: (0, 0, i, 0)),
            pl.BlockSpec((HID, HID), lambda i: (0, 0)),
            pl.BlockSpec((HID, HID), lambda i: (0, 0)),
            pl.BlockSpec((HID, HID), lambda i: (0, 0)),
            pl.BlockSpec((1, HID), lambda i: (0, 0)),
            pl.BlockSpec((1, HID), lambda i: (0, 0)),
        ],
        out_specs=pl.BlockSpec((9, NB, HID), lambda i: (0, i, 0)),
        out_shape=jax.ShapeDtypeStruct((9, N_PAD, HID), jnp.float32),
    )(xt, t, m, w3_0, w3_1, w3_2, bt1, bt3)


def kernel(X, edge_index, edge_weight, edge_attr,
           W1_0, b1_0, W1_1, b1_1, W1_2, b1_2,
           W3_0, b3_0, W3_1, b3_1, W3_2, b3_2,
           W2_0, b2_0, W2_1, b2_1, W2_2, b2_2):
    xt = jnp.transpose(X.reshape(N_NODES, HID, 9), (2, 0, 1))
    xt = jnp.pad(xt, ((0, 0), (0, N_PAD - N_NODES), (0, 0)))

    t = _node_table(xt, W1_0, W1_1, W1_2)

    wk = jnp.stack([W2_2[k::3, :] for k in range(3)])
    bk = jnp.stack([b2_2[k::3][None, :] for k in range(3)])
    b1k = jnp.stack([b1_0[None, :], b1_1[None, :], b1_2[None, :]])
    f0, f1, f2, fg = _edge_mlp(edge_attr, edge_weight[:, None],
                               W2_0, b2_0[None, :], W2_1, b2_1[None, :],
                               wk, bk, b1k)

    zeros = jnp.zeros((N_PAD, HID), jnp.float32)
    src_r = edge_index[1].reshape(E // W, W)
    dst_r = edge_index[0].reshape(E // W, W)
    m = _sc_messages(t, src_r, dst_r, f0, f1, f2, fg, zeros)

    bt1 = (b1_0 + b1_1 + b1_2)[None, :]
    bt3 = (b3_0 + b3_1 + b3_2)[None, :]
    xo = _finalize(xt, t, m, W3_0, W3_1, W3_2, bt1, bt3)

    xo = xo[:, :N_NODES, :]
    return jnp.transpose(xo, (1, 2, 0)).reshape(N_NODES, HID, 3, 3)


# speedup vs baseline: 1.0115x; 1.0104x over previous
import functools

import jax
import jax.numpy as jnp
from jax import lax
from jax.experimental import pallas as pl
from jax.experimental.pallas import tpu as pltpu
from jax.experimental.pallas import tpu_sc as plsc

N_NODES = 10001
E = 160000
HID = 128
CUTOFF_UPPER = 4.5

N_PAD = 10240
NB = 256
EB = 1600
W = 40
PER_TILE = E // 32
N_WIN = PER_TILE // W
GW = 5
N_GROUP = N_WIN // GW
RING = 3
N_STRIPE = N_PAD // 16


def _silu(x):
    return x * (1.0 / (1.0 + jnp.exp(-x)))


def _bdot(a, b):
    return lax.dot_general(a, b, (((1,), (1,)), ((), ())),
                           preferred_element_type=jnp.float32)


def _expand9(c, a, s, extra):
    return [
        c + s[0] + extra,
        a[0] + s[1] + extra,
        a[1] + s[2] + extra,
        -a[0] + s[1] + extra,
        c + s[3] + extra,
        a[2] + s[4] + extra,
        -a[1] + s[2] + extra,
        -a[2] + s[4] + extra,
        c - s[0] - s[3] + extra,
    ]


def _mat3mul(A, B):
    return [
        sum(A[3 * i + r] * B[3 * r + j] for r in range(3))
        for i in range(3) for j in range(3)
    ]


def _table_body(xt_ref, w0_ref, w1_ref, w2_ref, t_ref):
    x = [xt_ref[p] for p in range(9)]
    norm2 = sum(v * v for v in x) + 1.0
    inv = 1.0 / norm2
    x = [v * inv for v in x]
    ci = (x[0] + x[4] + x[8]) * (1.0 / 3.0)
    comps = [
        ci,
        0.5 * (x[1] - x[3]), 0.5 * (x[2] - x[6]), 0.5 * (x[5] - x[7]),
        x[0] - ci, 0.5 * (x[1] + x[3]), 0.5 * (x[2] + x[6]),
        x[4] - ci, 0.5 * (x[5] + x[7]),
    ]
    ws = [w0_ref[...], w1_ref[...], w2_ref[...]]
    wsel = [0, 1, 1, 1, 2, 2, 2, 2, 2]
    for p in range(9):
        t_ref[p] = _bdot(comps[p], ws[wsel[p]])


def _node_table(xt, w1_0, w1_1, w1_2):
    grid = N_PAD // NB
    return pl.pallas_call(
        _table_body,
        grid=(grid,),
        in_specs=[
            pl.BlockSpec((9, NB, HID), lambda i: (0, i, 0)),
            pl.BlockSpec((HID, HID), lambda i: (0, 0)),
            pl.BlockSpec((HID, HID), lambda i: (0, 0)),
            pl.BlockSpec((HID, HID), lambda i: (0, 0)),
        ],
        out_specs=pl.BlockSpec((9, NB, HID), lambda i: (0, i, 0)),
        out_shape=jax.ShapeDtypeStruct((9, N_PAD, HID), jnp.float32),
    )(xt, w1_0, w1_1, w1_2)


def _mlp_body(ea_ref, ew_ref, w0_ref, b0_ref, w1_ref, b1_ref,
              wk_ref, bk_ref, b1k_ref, f0_ref, f1_ref, f2_ref, g_ref):
    h = _silu(_bdot(ea_ref[...], w0_ref[...]) + b0_ref[...])
    h = _silu(_bdot(h, w1_ref[...]) + b1_ref[...])
    w = ew_ref[...]
    c = 0.5 * (jnp.cos(w * (jnp.pi / CUTOFF_UPPER)) + 1.0)
    c = jnp.where(w < CUTOFF_UPPER, c, 0.0)
    outs = [f0_ref, f1_ref, f2_ref]
    fks = []
    for k in range(3):
        fk = _silu(_bdot(h, wk_ref[k]) + bk_ref[k]) * c
        outs[k][...] = fk
        fks.append(fk)
    g_ref[...] = (fks[0] * b1k_ref[0] + fks[1] * b1k_ref[1]
                  + fks[2] * b1k_ref[2])


def _edge_mlp(edge_attr, edge_weight, w2_0, b2_0, w2_1, b2_1, wk, bk, b1k):
    grid = E // EB
    out = pl.pallas_call(
        _mlp_body,
        grid=(grid,),
        in_specs=[
            pl.BlockSpec((EB, 32), lambda i: (i, 0)),
            pl.BlockSpec((EB, 1), lambda i: (i, 0)),
            pl.BlockSpec((HID, 32), lambda i: (0, 0)),
            pl.BlockSpec((1, HID), lambda i: (0, 0)),
            pl.BlockSpec((2 * HID, HID), lambda i: (0, 0)),
            pl.BlockSpec((1, 2 * HID), lambda i: (0, 0)),
            pl.BlockSpec((3, HID, 2 * HID), lambda i: (0, 0, 0)),
            pl.BlockSpec((3, 1, HID), lambda i: (0, 0, 0)),
            pl.BlockSpec((3, 1, HID), lambda i: (0, 0, 0)),
        ],
        out_specs=[pl.BlockSpec((EB, HID), lambda i: (i, 0))] * 4,
        out_shape=[jax.ShapeDtypeStruct((E, HID), jnp.float32)] * 4,
    )(edge_attr, edge_weight, w2_0, b2_0, w2_1, b2_1, wk, bk, b1k)
    return out


def _sc_body(t, src_r, dst_r, f0, f1, f2, fg, zeros, out,
             srcall, dstall, rows, fac, acc,
             sem_ix, sem_g, sem_f, sem_s):
    c = lax.axis_index("c")
    s = lax.axis_index("s")
    e_base = c * (E // 2) + s * PER_TILE
    w_base = c * (E // 2 // W) + s * N_WIN

    def drain_one(slot):
        pltpu.make_async_copy(zeros.at[pl.ds(0, W)], rows.at[slot],
                              sem_s).wait()

    fsel = [f0, f1, f1, f1, f2, f2, f2, f2, f2, fg]

    for p in range(10):
        d1 = pltpu.async_copy(src_r.at[pl.ds(w_base, N_WIN)], srcall, sem_ix)
        d2 = pltpu.async_copy(dst_r.at[pl.ds(w_base, N_WIN)], dstall, sem_ix)

        @pl.when(s == 0)
        def _():
            pltpu.sync_copy(zeros, acc)

        d1.wait()
        d2.wait()
        plsc.subcore_barrier()

        def group(gi, carry):
            w0 = gi * GW

            @pl.when(gi > 0)
            def _():
                for r in range(RING):
                    drain_one(r)

            gds = []
            fds = []

            def fire(j):
                slot = j % RING
                e0 = e_base + (w0 + j) * W
                if p < 9:
                    gds.append(pltpu.async_copy(
                        t.at[p].at[srcall.at[w0 + j]], rows.at[slot], sem_g))
                    fds.append(pltpu.async_copy(
                        fsel[p].at[pl.ds(e0, W)], fac.at[slot], sem_f))
                else:
                    gds.append(pltpu.async_copy(
                        fsel[p].at[pl.ds(e0, W)], rows.at[slot], sem_g))

            for j in range(RING):
                fire(j)

            for j in range(GW):
                slot = j % RING
                gds[j].wait()
                if p < 9:
                    fds[j].wait()

                    def edge(e, _):
                        for q in range(8):
                            sl = pl.ds(16 * q, 16)
                            rows[slot, e, sl] = rows[slot, e, sl] * fac[slot, e, sl]
                        return 0

                    lax.fori_loop(0, W, edge, 0)
                pltpu.async_copy(rows.at[slot], acc.at[dstall.at[w0 + j]],
                                 sem_s, add=True)
                if j + RING < GW:
                    drain_one(slot)
                    fire(j + RING)
            return 0

        lax.fori_loop(0, N_GROUP, group, 0)
        for r in range(RING):
            drain_one(r)
        plsc.subcore_barrier()
        r0 = s * N_STRIPE
        pltpu.sync_copy(acc.at[pl.ds(r0, N_STRIPE)],
                        out.at[c].at[p].at[pl.ds(r0, N_STRIPE)])
        plsc.subcore_barrier()


def _sc_messages(t, src_r, dst_r, f0, f1, f2, fg, zeros):
    mesh = plsc.VectorSubcoreMesh(core_axis_name="c", subcore_axis_name="s")
    fn = pl.kernel(
        _sc_body,
        out_type=jax.ShapeDtypeStruct((2, 10, N_PAD, HID), jnp.float32),
        mesh=mesh,
        compiler_params=pltpu.CompilerParams(use_tc_tiling_on_sc=False),
        scratch_types=[
            pltpu.VMEM((N_WIN, W), jnp.int32),
            pltpu.VMEM((N_WIN, W), jnp.int32),
            pltpu.VMEM((RING, W, HID), jnp.float32),
            pltpu.VMEM((RING, W, HID), jnp.float32),
            pltpu.VMEM_SHARED((N_PAD, HID), jnp.float32),
            pltpu.SemaphoreType.DMA,
            pltpu.SemaphoreType.DMA,
            pltpu.SemaphoreType.DMA,
            pltpu.SemaphoreType.DMA,
        ],
    )
    return fn(t, src_r, dst_r, f0, f1, f2, fg, zeros)


def _final_body(xt_ref, t_ref, m_ref, w0_ref, w1_ref, w2_ref,
                bt1_ref, bt3_ref, out_ref):
    x = [xt_ref[p] for p in range(9)]
    norm2 = sum(v * v for v in x) + 1.0
    inv = 1.0 / norm2
    xn = [v * inv for v in x]

    bt1 = bt1_ref[...]
    t = [t_ref[p] for p in range(9)]
    Y = _expand9(t[0], t[1:4], t[4:9], bt1)

    m = [m_ref[0, p] + m_ref[1, p] for p in range(10)]
    M = _expand9(m[0], m[1:4], m[4:9], m[9])

    nf = [p + q for p, q in zip(_mat3mul(M, Y), _mat3mul(Y, M))]

    ci = (nf[0] + nf[4] + nf[8]) * (1.0 / 3.0)
    comps = [
        ci,
        0.5 * (nf[1] - nf[3]), 0.5 * (nf[2] - nf[6]), 0.5 * (nf[5] - nf[7]),
        nf[0] - ci, 0.5 * (nf[1] + nf[3]), 0.5 * (nf[2] + nf[6]),
        nf[4] - ci, 0.5 * (nf[5] + nf[7]),
    ]
    nrm = 1.0 / (sum(v * v for v in nf) + 1.0)
    comps = [v * nrm for v in comps]

    ws = [w0_ref[...], w1_ref[...], w2_ref[...]]
    wsel = [0, 1, 1, 1, 2, 2, 2, 2, 2]
    d = [_bdot(comps[p], ws[wsel[p]]) for p in range(9)]
    bt3 = bt3_ref[...]
    dX = _expand9(d[0], d[1:4], d[4:9], bt3)
    dX2 = _mat3mul(dX, dX)
    for p in range(9):
        out_ref[p] = xn[p] + dX[p] + dX2[p]


def _finalize(xt, t, m, w3_0, w3_1, w3_2, bt1, bt3):
    grid = N_PAD // NB
    return pl.pallas_call(
        _final_body,
        grid=(grid,),
        in_specs=[
            pl.BlockSpec((9, NB, HID), lambda i: (0, i, 0)),
            pl.BlockSpec((9, NB, HID), lambda i: (0, i, 0)),
            pl.BlockSpec((2, 10, NB, HID), lambda i: (0, 0, i, 0)),
            pl.BlockSpec((HID, HID), lambda i: (0, 0)),
            pl.BlockSpec((HID, HID), lambda i: (0, 0)),
            pl.BlockSpec((HID, HID), lambda i: (0, 0)),
            pl.BlockSpec((1, HID), lambda i: (0, 0)),
            pl.BlockSpec((1, HID), lambda i: (0, 0)),
        ],
        out_specs=pl.BlockSpec((9, NB, HID), lambda i: (0, i, 0)),
        out_shape=jax.ShapeDtypeStruct((9, N_PAD, HID), jnp.float32),
    )(xt, t, m, w3_0, w3_1, w3_2, bt1, bt3)


def kernel(X, edge_index, edge_weight, edge_attr,
           W1_0, b1_0, W1_1, b1_1, W1_2, b1_2,
           W3_0, b3_0, W3_1, b3_1, W3_2, b3_2,
           W2_0, b2_0, W2_1, b2_1, W2_2, b2_2):
    xt = jnp.transpose(X.reshape(N_NODES, HID, 9), (2, 0, 1))
    xt = jnp.pad(xt, ((0, 0), (0, N_PAD - N_NODES), (0, 0)))

    t = _node_table(xt, W1_0, W1_1, W1_2)

    wk = jnp.stack([W2_2[k::3, :] for k in range(3)])
    bk = jnp.stack([b2_2[k::3][None, :] for k in range(3)])
    b1k = jnp.stack([b1_0[None, :], b1_1[None, :], b1_2[None, :]])
    f0, f1, f2, fg = _edge_mlp(edge_attr, edge_weight[:, None],
                               W2_0, b2_0[None, :], W2_1, b2_1[None, :],
                               wk, bk, b1k)

    zeros = jnp.zeros((N_PAD, HID), jnp.float32)
    src_r = edge_index[1].reshape(E // W, W)
    dst_r = edge_index[0].reshape(E // W, W)
    m = _sc_messages(t, src_r, dst_r, f0, f1, f2, fg, zeros)

    bt1 = (b1_0 + b1_1 + b1_2)[None, :]
    bt3 = (b3_0 + b3_1 + b3_2)[None, :]
    xo = _finalize(xt, t, m, W3_0, W3_1, W3_2, bt1, bt3)

    xo = xo[:, :N_NODES, :]
    return jnp.transpose(xo, (1, 2, 0)).reshape(N_NODES, HID, 3, 3)
